# baseline (speedup 1.0000x reference)
import jax
import jax.numpy as jnp
from jax.experimental import pallas as pl

_NSTREAM = 2
_BM = 1024


def _router_block(*refs):
    x_refs = refs[:_NSTREAM]
    w_ref = refs[_NSTREAM]
    o_ref = refs[_NSTREAM + 1]
    w = w_ref[...].astype(jnp.bfloat16)
    for q in range(_NSTREAM):
        x = x_refs[q][...].astype(jnp.bfloat16)
        logits_t = jax.lax.dot_general(
            w, x,
            dimension_numbers=(((1,), (1,)), ((), ())),
            preferred_element_type=jnp.float32,
        )
        m = jnp.max(logits_t, axis=0, keepdims=True)
        e = jnp.exp(logits_t - m)
        o_ref[:, q * _BM:(q + 1) * _BM] = e / jnp.sum(e, axis=0, keepdims=True)


def kernel(inputs, W):
    M, K = inputs.shape
    E = W.shape[0]
    cols_per_step = _NSTREAM * _BM
    grid = (M // cols_per_step,)
    in_specs = [
        pl.BlockSpec((_BM, K), lambda i, q=q: (_NSTREAM * i + q, 0))
        for q in range(_NSTREAM)
    ]
    in_specs.append(pl.BlockSpec((E, K), lambda i: (0, 0)))
    probs_t = pl.pallas_call(
        _router_block,
        grid=grid,
        in_specs=in_specs,
        out_specs=pl.BlockSpec((E, cols_per_step), lambda i: (0, i)),
        out_shape=jax.ShapeDtypeStruct((E, M), jnp.float32),
    )(*([inputs] * _NSTREAM), W)
    return probs_t.T
